# baseline (speedup 1.0000x reference)
import functools

import numpy as np
import jax
import jax.numpy as jnp
from jax import lax
from jax.experimental import pallas as pl
from jax.experimental.pallas import tpu as pltpu
from jax.experimental.pallas import tpu_sc as plsc

NROW = 128
NCOL = 32768
BLK = 1024
NBLK = NCOL // BLK

_I = np.arange(BLK)
_U_NP = (_I[:, None] < _I[None, :]).astype(np.float32)
_V = np.arange(NROW)
_W_NP = (_V[:, None] > _V[None, :]).astype(np.float32)


def _tc_body(x_ref, u_ref, w_ref, a_ref, q_ref, rs_ref, carry_ref):
    b = pl.program_id(0)

    @pl.when(b == 0)
    def _():
        carry_ref[...] = jnp.zeros((NROW, 1), jnp.float32)

    x = x_ref[...]
    m = jnp.min(x, axis=0, keepdims=True)
    rows = lax.broadcasted_iota(jnp.int32, (NROW, BLK), 0)
    a = jnp.min(jnp.where(x == m, rows, NROW), axis=0, keepdims=True)
    a_ref[...] = jnp.reshape(a, (BLK,))

    onehot = (rows == a)
    b16 = onehot.astype(jnp.bfloat16)
    c = jax.lax.dot_general(b16, u_ref[...], (((1,), (0,)), ((), ())),
                            preferred_element_type=jnp.float32)
    carry = carry_ref[...]
    q = jnp.sum(jnp.where(onehot, c + carry, 0.0), axis=0, keepdims=True)
    q_ref[...] = jnp.reshape(q.astype(jnp.int32), (BLK,))
    r = c[:, BLK - 1:BLK] + b16[:, BLK - 1:BLK].astype(jnp.float32)
    new_carry = carry + r
    carry_ref[...] = new_carry
    rs = jnp.sum(w_ref[...] * new_carry, axis=0, keepdims=True)
    rs_ref[...] = jnp.reshape(rs.astype(jnp.int32), (NROW,))


_tc = pl.pallas_call(
    _tc_body,
    grid=(NBLK,),
    in_specs=[
        pl.BlockSpec((NROW, BLK), lambda i: (0, i)),
        pl.BlockSpec((BLK, BLK), lambda i: (0, 0)),
        pl.BlockSpec((NROW, NROW), lambda i: (0, 0)),
    ],
    out_specs=[
        pl.BlockSpec((BLK,), lambda i: (i,)),
        pl.BlockSpec((BLK,), lambda i: (i,)),
        pl.BlockSpec((NROW,), lambda i: (0,)),
    ],
    out_shape=[
        jax.ShapeDtypeStruct((NCOL,), jnp.int32),
        jax.ShapeDtypeStruct((NCOL,), jnp.int32),
        jax.ShapeDtypeStruct((NROW,), jnp.int32),
    ],
    scratch_shapes=[pltpu.VMEM((NROW, 1), jnp.float32)],
)


NT = 32
SEG = NCOL // NT
CH = 128
NCH = SEG // CH


def _sc_body(a_hbm, q_hbm, rs_hbm, out_hbm, a_v, q_v, rs_v, pos_v, val_v):
    c = lax.axis_index("c")
    s = lax.axis_index("s")
    base = (s * 2 + c) * SEG
    pltpu.sync_copy(a_hbm.at[pl.ds(base, SEG)], a_v)
    pltpu.sync_copy(q_hbm.at[pl.ds(base, SEG)], q_v)
    pltpu.sync_copy(rs_hbm, rs_v)

    def chunk(ch, carry):
        def group(i, carry2):
            g = ch * (CH // 16) + i
            av = a_v[pl.ds(g * 16, 16)]
            qv = q_v[pl.ds(g * 16, 16)]
            pos = qv + plsc.load_gather(rs_v, [av])
            vals = lax.iota(jnp.int32, 16) + (base + g * 16)
            pos_v.at[ch][pl.ds(i * 16, 16)] = pos
            val_v.at[ch][pl.ds(i * 16, 16)] = vals
            return carry2

        lax.fori_loop(0, CH // 16, group, 0, unroll=True)
        pltpu.sync_copy(val_v.at[ch], out_hbm.at[pos_v.at[ch]])
        return carry

    lax.fori_loop(0, NCH, chunk, 0, unroll=True)


@functools.cache
def _sc_kernel():
    return functools.partial(
        pl.kernel,
        out_type=jax.ShapeDtypeStruct((NCOL,), jnp.int32),
        mesh=plsc.VectorSubcoreMesh(core_axis_name="c", subcore_axis_name="s"),
        compiler_params=pltpu.CompilerParams(needs_layout_passes=False),
        scratch_types=[
            pltpu.VMEM((SEG,), jnp.int32),
            pltpu.VMEM((SEG,), jnp.int32),
            pltpu.VMEM((NROW,), jnp.int32),
            pltpu.VMEM((NCH, CH), jnp.int32),
            pltpu.VMEM((NCH, CH), jnp.int32),
        ],
    )(_sc_body)


@jax.jit
def kernel(x):
    a, q, rs = _tc(x, jnp.asarray(_U_NP, jnp.bfloat16), jnp.asarray(_W_NP))
    out = _sc_kernel()(a, q, rs)
    return jnp.reshape(out, (1, NCOL))


# speedup vs baseline: 1.9496x; 1.9496x over previous
import functools

import numpy as np
import jax
import jax.numpy as jnp
from jax import lax
from jax.experimental import pallas as pl
from jax.experimental.pallas import tpu as pltpu
from jax.experimental.pallas import tpu_sc as plsc

NROW = 128
NCOL = 32768
BLK = 1024
NBLK = NCOL // BLK

_I = np.arange(BLK)
_U_NP = (_I[:, None] < _I[None, :]).astype(np.float32)
_V = np.arange(NROW)
_W_NP = (_V[:, None] > _V[None, :]).astype(np.float32)


def _tc_body(x_ref, u_ref, w_ref, a_ref, q_ref, rs_ref, carry_ref):
    b = pl.program_id(0)

    @pl.when(b == 0)
    def _():
        carry_ref[...] = jnp.zeros((NROW, 1), jnp.float32)

    x = x_ref[...]
    m = jnp.min(x, axis=0, keepdims=True)
    rows = lax.broadcasted_iota(jnp.int32, (NROW, BLK), 0)
    a = jnp.min(jnp.where(x == m, rows, NROW), axis=0, keepdims=True)
    a_ref[...] = jnp.reshape(a, (BLK,))

    onehot = (rows == a)
    b16 = onehot.astype(jnp.bfloat16)
    c = jax.lax.dot_general(b16, u_ref[...], (((1,), (0,)), ((), ())),
                            preferred_element_type=jnp.float32)
    carry = carry_ref[...]
    q = jnp.sum(jnp.where(onehot, c + carry, 0.0), axis=0, keepdims=True)
    q_ref[...] = jnp.reshape(q.astype(jnp.int32), (BLK,))
    r = c[:, BLK - 1:BLK] + b16[:, BLK - 1:BLK].astype(jnp.float32)
    new_carry = carry + r
    carry_ref[...] = new_carry
    rs = jnp.sum(w_ref[...] * new_carry, axis=0, keepdims=True)
    rs_ref[...] = jnp.reshape(rs.astype(jnp.int32), (NROW,))


_tc = pl.pallas_call(
    _tc_body,
    grid=(NBLK,),
    in_specs=[
        pl.BlockSpec((NROW, BLK), lambda i: (0, i)),
        pl.BlockSpec((BLK, BLK), lambda i: (0, 0)),
        pl.BlockSpec((NROW, NROW), lambda i: (0, 0)),
    ],
    out_specs=[
        pl.BlockSpec((BLK,), lambda i: (i,)),
        pl.BlockSpec((BLK,), lambda i: (i,)),
        pl.BlockSpec((NROW,), lambda i: (0,)),
    ],
    out_shape=[
        jax.ShapeDtypeStruct((NCOL,), jnp.int32),
        jax.ShapeDtypeStruct((NCOL,), jnp.int32),
        jax.ShapeDtypeStruct((NROW,), jnp.int32),
    ],
    scratch_shapes=[pltpu.VMEM((NROW, 1), jnp.float32)],
)


NT = 16
SEG = NCOL // NT
CH = 128
NCH = SEG // CH


def _sc_body(a_hbm, q_hbm, rs_hbm, out_hbm, a_v, q_v, rs_v, pos_v, val_v, img):
    c = lax.axis_index("c")
    s = lax.axis_index("s")

    @pl.when(c == 0)
    def _():
        base = s * SEG
        pltpu.sync_copy(a_hbm.at[pl.ds(base, SEG)], a_v)
        pltpu.sync_copy(q_hbm.at[pl.ds(base, SEG)], q_v)
        pltpu.sync_copy(rs_hbm, rs_v)

        def chunk(ch, carry):
            def group(i, carry2):
                g = ch * (CH // 16) + i
                av = a_v[pl.ds(g * 16, 16)]
                qv = q_v[pl.ds(g * 16, 16)]
                pos = qv + plsc.load_gather(rs_v, [av])
                vals = lax.iota(jnp.int32, 16) + (base + g * 16)
                pos_v.at[ch][pl.ds(i * 16, 16)] = pos
                val_v.at[ch][pl.ds(i * 16, 16)] = vals
                return carry2

            lax.fori_loop(0, CH // 16, group, 0, unroll=True)
            pltpu.sync_copy(val_v.at[ch], img.at[pos_v.at[ch]])
            return carry

        lax.fori_loop(0, NCH, chunk, 0, unroll=True)
        plsc.subcore_barrier()
        pltpu.sync_copy(img.at[pl.ds(base, SEG)], out_hbm.at[pl.ds(base, SEG)])


@functools.cache
def _sc_kernel():
    return functools.partial(
        pl.kernel,
        out_type=jax.ShapeDtypeStruct((NCOL,), jnp.int32),
        mesh=plsc.VectorSubcoreMesh(core_axis_name="c", subcore_axis_name="s"),
        compiler_params=pltpu.CompilerParams(needs_layout_passes=False),
        scratch_types=[
            pltpu.VMEM((SEG,), jnp.int32),
            pltpu.VMEM((SEG,), jnp.int32),
            pltpu.VMEM((NROW,), jnp.int32),
            pltpu.VMEM((NCH, CH), jnp.int32),
            pltpu.VMEM((NCH, CH), jnp.int32),
            pltpu.VMEM_SHARED((NCOL,), jnp.int32),
        ],
    )(_sc_body)


@jax.jit
def kernel(x):
    a, q, rs = _tc(x, jnp.asarray(_U_NP, jnp.bfloat16), jnp.asarray(_W_NP))
    out = _sc_kernel()(a, q, rs)
    return jnp.reshape(out, (1, NCOL))


# speedup vs baseline: 2.1003x; 1.0773x over previous
import functools

import numpy as np
import jax
import jax.numpy as jnp
from jax import lax
from jax.experimental import pallas as pl
from jax.experimental.pallas import tpu as pltpu
from jax.experimental.pallas import tpu_sc as plsc

NROW = 128
NCOL = 32768
BLK = 1024
NBLK = NCOL // BLK

HB = 512
_I = np.arange(HB)
_U_NP = (_I[:, None] < _I[None, :]).astype(np.float32)
_V = np.arange(NROW)
_W_NP = (_V[:, None] > _V[None, :]).astype(np.float32)


def _tc_body(x_ref, u_ref, w_ref, a_ref, q_ref, rs_ref, carry_ref):
    b = pl.program_id(0)

    @pl.when(b == 0)
    def _():
        carry_ref[...] = jnp.zeros((NROW, 1), jnp.float32)

    rows = lax.broadcasted_iota(jnp.int32, (NROW, HB), 0)
    u = u_ref[...]
    carry = carry_ref[...]
    for h in range(BLK // HB):
        x = x_ref[:, pl.ds(h * HB, HB)]
        m = jnp.min(x, axis=0, keepdims=True)
        a = jnp.min(jnp.where(x == m, rows, NROW), axis=0, keepdims=True)
        a_ref[pl.ds(h * HB, HB)] = jnp.reshape(a, (HB,))

        onehot = (rows == a)
        b16 = onehot.astype(jnp.bfloat16)
        c = jax.lax.dot_general(b16, u, (((1,), (0,)), ((), ())),
                                preferred_element_type=jnp.float32)
        q = jnp.sum(jnp.where(onehot, c + carry, 0.0), axis=0, keepdims=True)
        q_ref[pl.ds(h * HB, HB)] = jnp.reshape(q.astype(jnp.int32), (HB,))
        r = c[:, HB - 1:HB] + b16[:, HB - 1:HB].astype(jnp.float32)
        carry = carry + r
    new_carry = carry
    carry_ref[...] = new_carry
    rs = jnp.sum(w_ref[...] * new_carry, axis=0, keepdims=True)
    rs_ref[...] = jnp.reshape(rs.astype(jnp.int32), (NROW,))


_tc = pl.pallas_call(
    _tc_body,
    grid=(NBLK,),
    in_specs=[
        pl.BlockSpec((NROW, BLK), lambda i: (0, i)),
        pl.BlockSpec((HB, HB), lambda i: (0, 0)),
        pl.BlockSpec((NROW, NROW), lambda i: (0, 0)),
    ],
    out_specs=[
        pl.BlockSpec((BLK,), lambda i: (i,)),
        pl.BlockSpec((BLK,), lambda i: (i,)),
        pl.BlockSpec((NROW,), lambda i: (0,)),
    ],
    out_shape=[
        jax.ShapeDtypeStruct((NCOL,), jnp.int32),
        jax.ShapeDtypeStruct((NCOL,), jnp.int32),
        jax.ShapeDtypeStruct((NROW,), jnp.int32),
    ],
    scratch_shapes=[pltpu.VMEM((NROW, 1), jnp.float32)],
)


NT = 16
SEG = NCOL // NT
CH = 128
NCH = SEG // CH


def _sc_body(a_hbm, q_hbm, rs_hbm, out_hbm, a_v, q_v, rs_v, pos_v, val_v, img):
    c = lax.axis_index("c")
    s = lax.axis_index("s")

    @pl.when(c == 0)
    def _():
        base = s * SEG
        pltpu.sync_copy(a_hbm.at[pl.ds(base, SEG)], a_v)
        pltpu.sync_copy(q_hbm.at[pl.ds(base, SEG)], q_v)
        pltpu.sync_copy(rs_hbm, rs_v)

        def chunk(ch, carry):
            def group(i, carry2):
                g = ch * (CH // 16) + i
                av = a_v[pl.ds(g * 16, 16)]
                qv = q_v[pl.ds(g * 16, 16)]
                pos = qv + plsc.load_gather(rs_v, [av])
                vals = lax.iota(jnp.int32, 16) + (base + g * 16)
                pos_v.at[ch][pl.ds(i * 16, 16)] = pos
                val_v.at[ch][pl.ds(i * 16, 16)] = vals
                return carry2

            lax.fori_loop(0, CH // 16, group, 0, unroll=True)
            pltpu.sync_copy(val_v.at[ch], img.at[pos_v.at[ch]])
            return carry

        lax.fori_loop(0, NCH, chunk, 0, unroll=True)
        plsc.subcore_barrier()
        pltpu.sync_copy(img.at[pl.ds(base, SEG)], out_hbm.at[pl.ds(base, SEG)])


@functools.cache
def _sc_kernel():
    return functools.partial(
        pl.kernel,
        out_type=jax.ShapeDtypeStruct((NCOL,), jnp.int32),
        mesh=plsc.VectorSubcoreMesh(core_axis_name="c", subcore_axis_name="s"),
        compiler_params=pltpu.CompilerParams(needs_layout_passes=False),
        scratch_types=[
            pltpu.VMEM((SEG,), jnp.int32),
            pltpu.VMEM((SEG,), jnp.int32),
            pltpu.VMEM((NROW,), jnp.int32),
            pltpu.VMEM((NCH, CH), jnp.int32),
            pltpu.VMEM((NCH, CH), jnp.int32),
            pltpu.VMEM_SHARED((NCOL,), jnp.int32),
        ],
    )(_sc_body)


@jax.jit
def kernel(x):
    a, q, rs = _tc(x, jnp.asarray(_U_NP, jnp.bfloat16), jnp.asarray(_W_NP))
    out = _sc_kernel()(a, q, rs)
    return jnp.reshape(out, (1, NCOL))


# speedup vs baseline: 2.1352x; 1.0166x over previous
import functools

import numpy as np
import jax
import jax.numpy as jnp
from jax import lax
from jax.experimental import pallas as pl
from jax.experimental.pallas import tpu as pltpu
from jax.experimental.pallas import tpu_sc as plsc

NROW = 128
NCOL = 32768
BLK = 1024
NBLK = NCOL // BLK

HB = 256
_I = np.arange(HB)
_U_NP = (_I[:, None] < _I[None, :]).astype(np.float32)
_V = np.arange(NROW)
_W_NP = (_V[:, None] > _V[None, :]).astype(np.float32)


def _tc_body(x_ref, u_ref, w_ref, a_ref, q_ref, rs_ref, carry_ref):
    b = pl.program_id(0)

    @pl.when(b == 0)
    def _():
        carry_ref[...] = jnp.zeros((NROW, 1), jnp.float32)

    rows = lax.broadcasted_iota(jnp.int32, (NROW, HB), 0)
    u = u_ref[...]
    carry = carry_ref[...]
    for h in range(BLK // HB):
        x = x_ref[:, pl.ds(h * HB, HB)]
        m = jnp.min(x, axis=0, keepdims=True)
        a = jnp.min(jnp.where(x == m, rows, NROW), axis=0, keepdims=True)
        a_ref[pl.ds(h * HB, HB)] = jnp.reshape(a, (HB,))

        onehot = (rows == a)
        b16 = onehot.astype(jnp.bfloat16)
        c = jax.lax.dot_general(b16, u, (((1,), (0,)), ((), ())),
                                preferred_element_type=jnp.float32)
        q = jnp.sum(jnp.where(onehot, c + carry, 0.0), axis=0, keepdims=True)
        q_ref[pl.ds(h * HB, HB)] = jnp.reshape(q.astype(jnp.int32), (HB,))
        r = c[:, HB - 1:HB] + b16[:, HB - 1:HB].astype(jnp.float32)
        carry = carry + r
    new_carry = carry
    carry_ref[...] = new_carry
    rs = jnp.sum(w_ref[...] * new_carry, axis=0, keepdims=True)
    rs_ref[...] = jnp.reshape(rs.astype(jnp.int32), (NROW,))


_tc = pl.pallas_call(
    _tc_body,
    grid=(NBLK,),
    in_specs=[
        pl.BlockSpec((NROW, BLK), lambda i: (0, i)),
        pl.BlockSpec((HB, HB), lambda i: (0, 0)),
        pl.BlockSpec((NROW, NROW), lambda i: (0, 0)),
    ],
    out_specs=[
        pl.BlockSpec((BLK,), lambda i: (i,)),
        pl.BlockSpec((BLK,), lambda i: (i,)),
        pl.BlockSpec((NROW,), lambda i: (0,)),
    ],
    out_shape=[
        jax.ShapeDtypeStruct((NCOL,), jnp.int32),
        jax.ShapeDtypeStruct((NCOL,), jnp.int32),
        jax.ShapeDtypeStruct((NROW,), jnp.int32),
    ],
    scratch_shapes=[pltpu.VMEM((NROW, 1), jnp.float32)],
)


NT = 16
SEG = NCOL // NT
CH = 128
NCH = SEG // CH


def _sc_body(a_hbm, q_hbm, rs_hbm, out_hbm, a_v, q_v, rs_v, pos_v, val_v, img):
    c = lax.axis_index("c")
    s = lax.axis_index("s")

    @pl.when(c == 0)
    def _():
        base = s * SEG
        pltpu.sync_copy(a_hbm.at[pl.ds(base, SEG)], a_v)
        pltpu.sync_copy(q_hbm.at[pl.ds(base, SEG)], q_v)
        pltpu.sync_copy(rs_hbm, rs_v)

        def chunk(ch, carry):
            def group(i, carry2):
                g = ch * (CH // 16) + i
                av = a_v[pl.ds(g * 16, 16)]
                qv = q_v[pl.ds(g * 16, 16)]
                pos = qv + plsc.load_gather(rs_v, [av])
                vals = lax.iota(jnp.int32, 16) + (base + g * 16)
                pos_v.at[ch][pl.ds(i * 16, 16)] = pos
                val_v.at[ch][pl.ds(i * 16, 16)] = vals
                return carry2

            lax.fori_loop(0, CH // 16, group, 0, unroll=True)
            pltpu.sync_copy(val_v.at[ch], img.at[pos_v.at[ch]])
            return carry

        lax.fori_loop(0, NCH, chunk, 0, unroll=True)
        plsc.subcore_barrier()
        pltpu.sync_copy(img.at[pl.ds(base, SEG)], out_hbm.at[pl.ds(base, SEG)])


@functools.cache
def _sc_kernel():
    return functools.partial(
        pl.kernel,
        out_type=jax.ShapeDtypeStruct((NCOL,), jnp.int32),
        mesh=plsc.VectorSubcoreMesh(core_axis_name="c", subcore_axis_name="s"),
        compiler_params=pltpu.CompilerParams(needs_layout_passes=False),
        scratch_types=[
            pltpu.VMEM((SEG,), jnp.int32),
            pltpu.VMEM((SEG,), jnp.int32),
            pltpu.VMEM((NROW,), jnp.int32),
            pltpu.VMEM((NCH, CH), jnp.int32),
            pltpu.VMEM((NCH, CH), jnp.int32),
            pltpu.VMEM_SHARED((NCOL,), jnp.int32),
        ],
    )(_sc_body)


@jax.jit
def kernel(x):
    a, q, rs = _tc(x, jnp.asarray(_U_NP, jnp.bfloat16), jnp.asarray(_W_NP))
    out = _sc_kernel()(a, q, rs)
    return jnp.reshape(out, (1, NCOL))


# speedup vs baseline: 2.6386x; 1.2358x over previous
import functools

import numpy as np
import jax
import jax.numpy as jnp
from jax import lax
from jax.experimental import pallas as pl
from jax.experimental.pallas import tpu as pltpu
from jax.experimental.pallas import tpu_sc as plsc

NROW = 128
NCOL = 32768
BLK = 2048
NBLK = NCOL // BLK

HB = 256
_I = np.arange(HB)
_U_NP = (_I[:, None] < _I[None, :]).astype(np.float32)
_V = np.arange(NROW)
_W_NP = (_V[:, None] > _V[None, :]).astype(np.float32)


def _tc_body(x_ref, u_ref, w_ref, a_ref, q_ref, rs_ref, carry_ref):
    b = pl.program_id(0)

    @pl.when(b == 0)
    def _():
        carry_ref[...] = jnp.zeros((NROW, 1), jnp.float32)

    rows = lax.broadcasted_iota(jnp.int32, (NROW, HB), 0)
    u = u_ref[...]
    carry = carry_ref[...]
    for h in range(BLK // HB):
        x = x_ref[:, pl.ds(h * HB, HB)]
        m = jnp.min(x, axis=0, keepdims=True)
        a = jnp.min(jnp.where(x == m, rows, NROW), axis=0, keepdims=True)
        a_ref[pl.ds(h * HB, HB)] = jnp.reshape(a, (HB,))

        onehot = (rows == a)
        b16 = onehot.astype(jnp.bfloat16)
        c = jax.lax.dot_general(b16, u, (((1,), (0,)), ((), ())),
                                preferred_element_type=jnp.float32)
        q = jnp.sum(jnp.where(onehot, c + carry, 0.0), axis=0, keepdims=True)
        q_ref[pl.ds(h * HB, HB)] = jnp.reshape(q.astype(jnp.int32), (HB,))
        r = c[:, HB - 1:HB] + b16[:, HB - 1:HB].astype(jnp.float32)
        carry = carry + r
    new_carry = carry
    carry_ref[...] = new_carry
    rs = jnp.sum(w_ref[...] * new_carry, axis=0, keepdims=True)
    rs_ref[...] = jnp.reshape(rs.astype(jnp.int32), (NROW,))


_tc = pl.pallas_call(
    _tc_body,
    grid=(NBLK,),
    in_specs=[
        pl.BlockSpec((NROW, BLK), lambda i: (0, i)),
        pl.BlockSpec((HB, HB), lambda i: (0, 0)),
        pl.BlockSpec((NROW, NROW), lambda i: (0, 0)),
    ],
    out_specs=[
        pl.BlockSpec((BLK,), lambda i: (i,)),
        pl.BlockSpec((BLK,), lambda i: (i,)),
        pl.BlockSpec((NROW,), lambda i: (0,)),
    ],
    out_shape=[
        jax.ShapeDtypeStruct((NCOL,), jnp.int32),
        jax.ShapeDtypeStruct((NCOL,), jnp.int32),
        jax.ShapeDtypeStruct((NROW,), jnp.int32),
    ],
    scratch_shapes=[pltpu.VMEM((NROW, 1), jnp.float32)],
)


NT = 16
SEG = NCOL // NT
CH = 128
NCH = SEG // CH


def _sc_body(a_hbm, q_hbm, rs_hbm, out_hbm, a_v, q_v, rs_v, pos_v, val_v, img):
    c = lax.axis_index("c")
    s = lax.axis_index("s")

    @pl.when(c == 0)
    def _():
        base = s * SEG
        pltpu.sync_copy(a_hbm.at[pl.ds(base, SEG)], a_v)
        pltpu.sync_copy(q_hbm.at[pl.ds(base, SEG)], q_v)
        pltpu.sync_copy(rs_hbm, rs_v)

        def chunk(ch, carry):
            def group(i, carry2):
                g = ch * (CH // 16) + i
                av = a_v[pl.ds(g * 16, 16)]
                qv = q_v[pl.ds(g * 16, 16)]
                pos = qv + plsc.load_gather(rs_v, [av])
                vals = lax.iota(jnp.int32, 16) + (base + g * 16)
                pos_v.at[ch][pl.ds(i * 16, 16)] = pos
                val_v.at[ch][pl.ds(i * 16, 16)] = vals
                return carry2

            lax.fori_loop(0, CH // 16, group, 0, unroll=True)
            pltpu.sync_copy(val_v.at[ch], img.at[pos_v.at[ch]])
            return carry

        lax.fori_loop(0, NCH, chunk, 0, unroll=True)
        plsc.subcore_barrier()
        pltpu.sync_copy(img.at[pl.ds(base, SEG)], out_hbm.at[pl.ds(base, SEG)])


@functools.cache
def _sc_kernel():
    return functools.partial(
        pl.kernel,
        out_type=jax.ShapeDtypeStruct((NCOL,), jnp.int32),
        mesh=plsc.VectorSubcoreMesh(core_axis_name="c", subcore_axis_name="s"),
        compiler_params=pltpu.CompilerParams(needs_layout_passes=False),
        scratch_types=[
            pltpu.VMEM((SEG,), jnp.int32),
            pltpu.VMEM((SEG,), jnp.int32),
            pltpu.VMEM((NROW,), jnp.int32),
            pltpu.VMEM((NCH, CH), jnp.int32),
            pltpu.VMEM((NCH, CH), jnp.int32),
            pltpu.VMEM_SHARED((NCOL,), jnp.int32),
        ],
    )(_sc_body)


@jax.jit
def kernel(x):
    a, q, rs = _tc(x, jnp.asarray(_U_NP, jnp.bfloat16), jnp.asarray(_W_NP))
    out = _sc_kernel()(a, q, rs)
    return jnp.reshape(out, (1, NCOL))


# speedup vs baseline: 2.8905x; 1.0954x over previous
import functools

import numpy as np
import jax
import jax.numpy as jnp
from jax import lax
from jax.experimental import pallas as pl
from jax.experimental.pallas import tpu as pltpu
from jax.experimental.pallas import tpu_sc as plsc

NROW = 128
NCOL = 32768
BLK = 4096
NBLK = NCOL // BLK

HB = 256
_I = np.arange(HB)
_U_NP = (_I[:, None] < _I[None, :]).astype(np.float32)
_V = np.arange(NROW)
_W_NP = (_V[:, None] > _V[None, :]).astype(np.float32)


def _tc_body(x_ref, u_ref, w_ref, a_ref, q_ref, rs_ref, carry_ref):
    b = pl.program_id(0)

    @pl.when(b == 0)
    def _():
        carry_ref[...] = jnp.zeros((NROW, 1), jnp.float32)

    rows = lax.broadcasted_iota(jnp.int32, (NROW, HB), 0)
    u = u_ref[...]
    carry = carry_ref[...]
    for h in range(BLK // HB):
        x = x_ref[:, pl.ds(h * HB, HB)]
        m = jnp.min(x, axis=0, keepdims=True)
        a = jnp.min(jnp.where(x == m, rows, NROW), axis=0, keepdims=True)
        a_ref[pl.ds(h * HB, HB)] = jnp.reshape(a, (HB,))

        onehot = (rows == a)
        b16 = onehot.astype(jnp.bfloat16)
        c = jax.lax.dot_general(b16, u, (((1,), (0,)), ((), ())),
                                preferred_element_type=jnp.float32)
        q = jnp.sum(jnp.where(onehot, c + carry, 0.0), axis=0, keepdims=True)
        q_ref[pl.ds(h * HB, HB)] = jnp.reshape(q.astype(jnp.int32), (HB,))
        r = c[:, HB - 1:HB] + b16[:, HB - 1:HB].astype(jnp.float32)
        carry = carry + r
    new_carry = carry
    carry_ref[...] = new_carry
    rs = jnp.sum(w_ref[...] * new_carry, axis=0, keepdims=True)
    rs_ref[...] = jnp.reshape(rs.astype(jnp.int32), (NROW,))


_tc = pl.pallas_call(
    _tc_body,
    grid=(NBLK,),
    in_specs=[
        pl.BlockSpec((NROW, BLK), lambda i: (0, i)),
        pl.BlockSpec((HB, HB), lambda i: (0, 0)),
        pl.BlockSpec((NROW, NROW), lambda i: (0, 0)),
    ],
    out_specs=[
        pl.BlockSpec((BLK,), lambda i: (i,)),
        pl.BlockSpec((BLK,), lambda i: (i,)),
        pl.BlockSpec((NROW,), lambda i: (0,)),
    ],
    out_shape=[
        jax.ShapeDtypeStruct((NCOL,), jnp.int32),
        jax.ShapeDtypeStruct((NCOL,), jnp.int32),
        jax.ShapeDtypeStruct((NROW,), jnp.int32),
    ],
    scratch_shapes=[pltpu.VMEM((NROW, 1), jnp.float32)],
)


NT = 16
SEG = NCOL // NT
CH = 128
NCH = SEG // CH


def _sc_body(a_hbm, q_hbm, rs_hbm, out_hbm, a_v, q_v, rs_v, pos_v, val_v, img):
    c = lax.axis_index("c")
    s = lax.axis_index("s")

    @pl.when(c == 0)
    def _():
        base = s * SEG
        pltpu.sync_copy(a_hbm.at[pl.ds(base, SEG)], a_v)
        pltpu.sync_copy(q_hbm.at[pl.ds(base, SEG)], q_v)
        pltpu.sync_copy(rs_hbm, rs_v)

        def chunk(ch, carry):
            def group(i, carry2):
                g = ch * (CH // 16) + i
                av = a_v[pl.ds(g * 16, 16)]
                qv = q_v[pl.ds(g * 16, 16)]
                pos = qv + plsc.load_gather(rs_v, [av])
                vals = lax.iota(jnp.int32, 16) + (base + g * 16)
                pos_v.at[ch][pl.ds(i * 16, 16)] = pos
                val_v.at[ch][pl.ds(i * 16, 16)] = vals
                return carry2

            lax.fori_loop(0, CH // 16, group, 0, unroll=True)
            pltpu.sync_copy(val_v.at[ch], img.at[pos_v.at[ch]])
            return carry

        lax.fori_loop(0, NCH, chunk, 0, unroll=True)
        plsc.subcore_barrier()
        pltpu.sync_copy(img.at[pl.ds(base, SEG)], out_hbm.at[pl.ds(base, SEG)])


@functools.cache
def _sc_kernel():
    return functools.partial(
        pl.kernel,
        out_type=jax.ShapeDtypeStruct((NCOL,), jnp.int32),
        mesh=plsc.VectorSubcoreMesh(core_axis_name="c", subcore_axis_name="s"),
        compiler_params=pltpu.CompilerParams(needs_layout_passes=False),
        scratch_types=[
            pltpu.VMEM((SEG,), jnp.int32),
            pltpu.VMEM((SEG,), jnp.int32),
            pltpu.VMEM((NROW,), jnp.int32),
            pltpu.VMEM((NCH, CH), jnp.int32),
            pltpu.VMEM((NCH, CH), jnp.int32),
            pltpu.VMEM_SHARED((NCOL,), jnp.int32),
        ],
    )(_sc_body)


@jax.jit
def kernel(x):
    a, q, rs = _tc(x, jnp.asarray(_U_NP, jnp.bfloat16), jnp.asarray(_W_NP))
    out = _sc_kernel()(a, q, rs)
    return jnp.reshape(out, (1, NCOL))
